# baseline (speedup 1.0000x reference)
import functools

import jax
import jax.numpy as jnp
from jax import lax
from jax.experimental import pallas as pl
from jax.experimental.pallas import tpu as pltpu
from jax.experimental.pallas import tpu_sc as plsc

VOCAB = 100000
EMBED = 16
BATCH = 1024
BAG = 20

NUM_CORES = 2
NUM_SUBCORES = 16
NUM_WORKERS = NUM_CORES * NUM_SUBCORES
B_PER_W = BATCH // NUM_WORKERS

VT = 2048
NV = (VOCAB + VT - 1) // VT


def _pool_body(idx_hbm, table_hbm, out_hbm, idx_v, rows_v, pooled_v, sem):
    wid = lax.axis_index("s") * NUM_CORES + lax.axis_index("c")
    base = wid * B_PER_W
    pltpu.sync_copy(idx_hbm.at[pl.ds(base * BAG, B_PER_W * BAG)], idx_v)
    pltpu.async_copy(table_hbm.at[idx_v], rows_v, sem).wait()
    for i in range(B_PER_W):
        r = rows_v[i * BAG, :]
        for j in range(1, BAG):
            r = r + rows_v[i * BAG + j, :]
        pooled_v[i, :] = r * (1.0 / BAG)
    pltpu.sync_copy(pooled_v, out_hbm.at[pl.ds(base, B_PER_W)])


_pool = functools.partial(
    pl.kernel,
    out_type=jax.ShapeDtypeStruct((BATCH, EMBED), jnp.float32),
    mesh=plsc.VectorSubcoreMesh(core_axis_name="c", subcore_axis_name="s"),
    scratch_types=[
        pltpu.VMEM((B_PER_W * BAG,), jnp.int32),
        pltpu.VMEM((B_PER_W * BAG, EMBED), jnp.float32),
        pltpu.VMEM((B_PER_W, EMBED), jnp.float32),
        pltpu.SemaphoreType.DMA,
    ],
    compiler_params=pltpu.CompilerParams(use_tc_tiling_on_sc=False),
)(_pool_body)


def _proj_body(pooled_ref, wt_ref, b_ref, out_ref):
    out_ref[...] = (
        jnp.dot(pooled_ref[...], wt_ref[...], preferred_element_type=jnp.float32)
        + b_ref[...]
    )


_proj = pl.pallas_call(
    _proj_body,
    grid=(NV,),
    in_specs=[
        pl.BlockSpec((BATCH, EMBED), lambda v: (0, 0)),
        pl.BlockSpec((EMBED, VT), lambda v: (0, v)),
        pl.BlockSpec((1, VT), lambda v: (0, v)),
    ],
    out_specs=pl.BlockSpec((BATCH, VT), lambda v: (0, v)),
    out_shape=jax.ShapeDtypeStruct((BATCH, VOCAB), jnp.float32),
    compiler_params=pltpu.CompilerParams(dimension_semantics=("parallel",)),
)


def kernel(inputs, emb_table, W, b):
    idx_flat = inputs.reshape(-1).astype(jnp.int32)
    pooled = _pool(idx_flat, emb_table)
    return _proj(pooled, W.T, b.reshape(1, VOCAB))
